# Initial kernel scaffold; baseline (speedup 1.0000x reference)
import jax
import jax.numpy as jnp
from jax.experimental import pallas as pl


def kernel(node_ids, embedding_weight):
    raise NotImplementedError("write your pallas kernel here")


# speedup vs baseline: 1.8757x; 1.8757x over previous
import functools

import jax
import jax.numpy as jnp
from jax import lax
from jax.experimental import pallas as pl
from jax.experimental.pallas import tpu as pltpu
from jax.experimental.pallas import tpu_sc as plsc

_D = 64
_RPD = 128
_K = 4
_GROUP = _K * _RPD


@functools.lru_cache(maxsize=None)
def _build(B, nw):
    b_per_w = B // nw
    n_dma = b_per_w // _RPD
    ng = n_dma // _K

    mesh = plsc.VectorSubcoreMesh(core_axis_name="c", subcore_axis_name="s")

    @functools.partial(
        pl.kernel,
        mesh=mesh,
        compiler_params=pltpu.CompilerParams(use_tc_tiling_on_sc=False),
        out_type=jax.ShapeDtypeStruct((B, _D), jnp.float32),
        scratch_types=[
            pltpu.VMEM((n_dma, _RPD), jnp.int32),
            pltpu.VMEM((2, _GROUP, _D), jnp.float32),
            pltpu.SemaphoreType.DMA,
            pltpu.SemaphoreType.DMA,
        ],
    )
    def gather_kernel(idx_hbm, table_hbm, out_hbm, idx_v, rows_v, sem0, sem1):
        wid = lax.axis_index("s") * 2 + lax.axis_index("c")
        base = wid * b_per_w
        pltpu.sync_copy(idx_hbm.at[wid], idx_v)

        sems = (sem0, sem1)

        def fire(g, b):
            for jj in range(_K):
                pltpu.async_copy(
                    table_hbm.at[idx_v.at[g * _K + jj]],
                    rows_v.at[b, pl.ds(jj * _RPD, _RPD)],
                    sems[b],
                )

        def drain(b):
            pltpu.make_async_copy(
                out_hbm.at[pl.ds(0, _GROUP)],
                rows_v.at[b],
                sems[b],
            ).wait()

        def store(g, b):
            pltpu.sync_copy(
                rows_v.at[b],
                out_hbm.at[pl.ds(base + g * _GROUP, _GROUP)],
            )

        fire(0, 0)

        def body(i, carry):
            for b in range(2):
                g = i * 2 + b

                @pl.when(g + 1 < ng)
                def _():
                    fire(g + 1, 1 - b)

                drain(b)
                store(g, b)
            return carry

        lax.fori_loop(0, ng // 2, body, 0)

    return gather_kernel


def kernel(node_ids, embedding_weight):
    r, c = node_ids.shape
    B = r * c
    info = plsc.get_sparse_core_info()
    nw = info.num_cores * info.num_subcores
    idx3 = node_ids.astype(jnp.int32).reshape(nw, (B // nw) // _RPD, _RPD)
    out = _build(B, nw)(idx3, embedding_weight)
    return out.reshape(r, c, _D)


# speedup vs baseline: 1.8770x; 1.0007x over previous
import functools

import jax
import jax.numpy as jnp
from jax import lax
from jax.experimental import pallas as pl
from jax.experimental.pallas import tpu as pltpu
from jax.experimental.pallas import tpu_sc as plsc

_D = 64
_RPD = 128
_K = 4
_GROUP = _K * _RPD


@functools.lru_cache(maxsize=None)
def _build(n_rows, n_cols, nw):
    rows_per_w = n_rows // nw

    mesh = plsc.VectorSubcoreMesh(core_axis_name="c", subcore_axis_name="s")

    @functools.partial(
        pl.kernel,
        mesh=mesh,
        compiler_params=pltpu.CompilerParams(use_tc_tiling_on_sc=False),
        out_type=jax.ShapeDtypeStruct((n_rows, n_cols, _D), jnp.float32),
        scratch_types=[
            pltpu.VMEM((n_cols, _GROUP), jnp.int32),
            pltpu.VMEM((2, _GROUP, _D), jnp.float32),
            pltpu.SemaphoreType.DMA,
            pltpu.SemaphoreType.DMA,
        ],
    )
    def gather_kernel(idx_hbm, table_hbm, out_hbm, idx_v, rows_v, sem0, sem1):
        wid = lax.axis_index("s") * 2 + lax.axis_index("c")
        base = wid * rows_per_w
        pltpu.sync_copy(idx_hbm.at[:, pl.ds(base, _GROUP)], idx_v)

        sems = (sem0, sem1)

        def fire(g, b):
            for jj in range(_K):
                pltpu.async_copy(
                    table_hbm.at[idx_v.at[g, pl.ds(jj * _RPD, _RPD)]],
                    rows_v.at[b, pl.ds(jj * _RPD, _RPD)],
                    sems[b],
                )

        def drain(b):
            pltpu.make_async_copy(
                table_hbm.at[pl.ds(0, _GROUP)],
                rows_v.at[b],
                sems[b],
            ).wait()

        def store(g, b):
            pltpu.sync_copy(
                rows_v.at[b],
                out_hbm.at[pl.ds(base, _GROUP), g],
            )

        fire(0, 0)

        def body(i, carry):
            for b in range(2):
                g = i * 2 + b

                @pl.when(g + 1 < n_cols)
                def _():
                    fire(g + 1, 1 - b)

                drain(b)
                store(g, b)
            return carry

        lax.fori_loop(0, n_cols // 2, body, 0)

    return gather_kernel


def kernel(node_ids, embedding_weight):
    r, c = node_ids.shape
    info = plsc.get_sparse_core_info()
    nw = info.num_cores * info.num_subcores
    ids_t = node_ids.astype(jnp.int32).T
    return _build(r, c, nw)(ids_t, embedding_weight)


# speedup vs baseline: 2.3310x; 1.2418x over previous
import functools

import jax
import jax.numpy as jnp
from jax import lax
from jax.experimental import pallas as pl
from jax.experimental.pallas import tpu as pltpu
from jax.experimental.pallas import tpu_sc as plsc

_D = 64
_DP = 128
_RPD = 128
_K = 2
_GROUP = _K * _RPD


@functools.lru_cache(maxsize=None)
def _build(n_rows, n_cols, nw):
    rows_per_w = n_rows // nw
    gpc = rows_per_w // _GROUP
    ng = n_cols * gpc

    mesh = plsc.VectorSubcoreMesh(core_axis_name="c", subcore_axis_name="s")

    @functools.partial(
        pl.kernel,
        mesh=mesh,
        compiler_params=pltpu.CompilerParams(use_tc_tiling_on_sc=False),
        out_type=jax.ShapeDtypeStruct((n_rows, (n_cols + 7) // 8, 8, _DP),
                                      jnp.float32),
        scratch_types=[
            pltpu.VMEM((n_cols, rows_per_w), jnp.int32),
            pltpu.VMEM((2, _GROUP, _DP), jnp.float32),
            pltpu.SemaphoreType.DMA,
            pltpu.SemaphoreType.DMA,
        ],
    )
    def gather_kernel(idx_hbm, table_hbm, out_hbm, idx_v, rows_v, sem0, sem1):
        wid = lax.axis_index("s") * 2 + lax.axis_index("c")
        base = wid * rows_per_w
        pltpu.sync_copy(idx_hbm.at[:, pl.ds(base, rows_per_w)], idx_v)

        sems = (sem0, sem1)

        def fire(g, b):
            j = g // gpc
            half = g % gpc
            for jj in range(_K):
                pltpu.async_copy(
                    table_hbm.at[
                        idx_v.at[j, pl.ds(half * _GROUP + jj * _RPD, _RPD)]],
                    rows_v.at[b, pl.ds(jj * _RPD, _RPD)],
                    sems[b],
                )

        def drain(b):
            pltpu.make_async_copy(
                table_hbm.at[pl.ds(0, _GROUP)],
                rows_v.at[b],
                sems[b],
            ).wait()

        def store(g, b):
            j = g // gpc
            half = g % gpc
            pltpu.sync_copy(
                rows_v.at[b],
                out_hbm.at[pl.ds(base + half * _GROUP, _GROUP), j // 8, j % 8],
            )

        fire(0, 0)

        def body(i, carry):
            for b in range(2):
                g = i * 2 + b

                @pl.when(g + 1 < ng)
                def _():
                    fire(g + 1, 1 - b)

                drain(b)
                store(g, b)
            return carry

        lax.fori_loop(0, ng // 2, body, 0)

    return gather_kernel


def kernel(node_ids, embedding_weight):
    r, c = node_ids.shape
    d = embedding_weight.shape[1]
    info = plsc.get_sparse_core_info()
    nw = info.num_cores * info.num_subcores
    ids_t = node_ids.astype(jnp.int32).T
    tab128 = jnp.pad(embedding_weight, ((0, 0), (0, _DP - d)))
    out4 = _build(r, c, nw)(ids_t, tab128)
    return out4.reshape(r, 8 * ((c + 7) // 8), _DP)[:, :c, :d]
